# baseline (speedup 1.0000x reference)
import functools

import jax
import jax.numpy as jnp
from jax import lax
from jax.experimental import pallas as pl
from jax.experimental.pallas import tpu as pltpu
from jax.experimental.pallas import tpu_sc as plsc

_CHUNK = 24


def kernel(patch, W):
    n = patch.shape[1] + 1
    d = W.shape[1]
    full_workers = n // _CHUNK
    rem_base = full_workers * _CHUNK
    rem = n - rem_base

    mesh = plsc.VectorSubcoreMesh(core_axis_name="c", subcore_axis_name="s")
    nc = 2

    @functools.partial(
        pl.kernel,
        mesh=mesh,
        out_type=jax.ShapeDtypeStruct((n, d), W.dtype),
    )
    def sc_copy(w_hbm, o_hbm):
        wid = lax.axis_index("s") * nc + lax.axis_index("c")

        @pl.when(wid < full_workers)
        def _():
            base = wid * _CHUNK
            pltpu.sync_copy(
                w_hbm.at[pl.ds(base, _CHUNK)], o_hbm.at[pl.ds(base, _CHUNK)]
            )

        @pl.when(wid == full_workers)
        def _():
            pltpu.sync_copy(
                w_hbm.at[pl.ds(rem_base, rem)], o_hbm.at[pl.ds(rem_base, rem)]
            )

    out = sc_copy(W)
    return out[None]


# speedup vs baseline: 1.2859x; 1.2859x over previous
import jax
import jax.numpy as jnp
from jax.experimental import pallas as pl
from jax.experimental.pallas import tpu as pltpu


def kernel(patch, W):
    n = patch.shape[1] + 1
    d = W.shape[1]
    n_pad = (n + 7) // 8 * 8

    def body(w_ref, o_ref, sem):
        cp = pltpu.make_async_copy(w_ref.at[pl.ds(0, n_pad)], o_ref, sem)
        cp.start()
        cp.wait()

    out = pl.pallas_call(
        body,
        out_shape=jax.ShapeDtypeStruct((n_pad, d), W.dtype),
        in_specs=[pl.BlockSpec(memory_space=pltpu.MemorySpace.HBM)],
        out_specs=pl.BlockSpec(memory_space=pltpu.MemorySpace.HBM),
        scratch_shapes=[pltpu.SemaphoreType.DMA],
    )(W)
    return out[:n][None]


# speedup vs baseline: 10.9104x; 8.4847x over previous
import jax
import jax.numpy as jnp
from jax.experimental import pallas as pl
from jax.experimental.pallas import tpu as pltpu

_BLOCK = 296


def kernel(patch, W):
    n = patch.shape[1] + 1
    d = W.shape[1]
    steps = (n + _BLOCK - 1) // _BLOCK

    def body(w_ref, o_ref):
        o_ref[0, ...] = w_ref[...]

    out = pl.pallas_call(
        body,
        out_shape=jax.ShapeDtypeStruct((1, n, d), W.dtype),
        grid=(steps,),
        in_specs=[pl.BlockSpec((_BLOCK, d), lambda i: (i, 0))],
        out_specs=pl.BlockSpec((1, _BLOCK, d), lambda i: (0, i, 0)),
        compiler_params=pltpu.CompilerParams(
            dimension_semantics=("parallel",),
        ),
    )(W)
    return out
